# baseline (speedup 1.0000x reference)
import jax
import jax.numpy as jnp
from jax.experimental import pallas as pl
from jax.experimental.pallas import tpu as pltpu

_ROWS = 100000
_CHUNK = 2000
_NCHUNKS = _ROWS // _CHUNK
_N_BUF = 12
_LEAD = 6


def _in_copy(in_hbm, bufs, in_sems, i):
    b = i % _N_BUF
    return pltpu.make_async_copy(
        in_hbm.at[pl.ds(i * _CHUNK, _CHUNK), :], bufs.at[b], in_sems.at[b])


def _out_copy(out_hbm, bufs, out_sems, k):
    b = k % _N_BUF
    return pltpu.make_async_copy(
        bufs.at[b], out_hbm.at[pl.ds(k * _CHUNK, _CHUNK), :], out_sems.at[b])


def _pipe_copy(in_hbm, out_hbm, bufs, in_sems, out_sems):
    for i in range(_NCHUNKS + _LEAD):
        if i < _NCHUNKS:
            if i >= _N_BUF:
                _out_copy(out_hbm, bufs, out_sems, i - _N_BUF).wait()
            _in_copy(in_hbm, bufs, in_sems, i).start()
        k = i - _LEAD
        if k >= 0:
            _in_copy(in_hbm, bufs, in_sems, k).wait()
            _out_copy(out_hbm, bufs, out_sems, k).start()
    for k in range(max(0, _NCHUNKS - _N_BUF), _NCHUNKS):
        _out_copy(out_hbm, bufs, out_sems, k).wait()


def _copy3_kernel(c_in, n_in, u_in, c_out, n_out, u_out,
                  buf128, buf64, in_sems, out_sems):
    _pipe_copy(c_in, c_out, buf128, in_sems, out_sems)
    _pipe_copy(n_in, n_out, buf128, in_sems, out_sems)
    _pipe_copy(u_in, u_out, buf64, in_sems, out_sems)


def kernel(c_embeddings, n_embeddings, u_embeddings):
    out = pl.pallas_call(
        _copy3_kernel,
        in_specs=[pl.BlockSpec(memory_space=pl.ANY)] * 3,
        out_specs=[pl.BlockSpec(memory_space=pl.ANY)] * 3,
        out_shape=(
            jax.ShapeDtypeStruct(c_embeddings.shape, c_embeddings.dtype),
            jax.ShapeDtypeStruct(n_embeddings.shape, n_embeddings.dtype),
            jax.ShapeDtypeStruct(u_embeddings.shape, u_embeddings.dtype),
        ),
        scratch_shapes=[
            pltpu.MemorySpace.VMEM((_N_BUF, _CHUNK, 128), jnp.float32),
            pltpu.MemorySpace.VMEM((_N_BUF, _CHUNK, 64), jnp.float32),
            pltpu.SemaphoreType.DMA((_N_BUF,)),
            pltpu.SemaphoreType.DMA((_N_BUF,)),
        ],
    )(c_embeddings, n_embeddings, u_embeddings)
    return (out[0], out[1], out[2])
